# baseline (speedup 1.0000x reference)
import functools

import jax
import jax.numpy as jnp
from jax import lax
from jax.experimental import pallas as pl
from jax.experimental.pallas import tpu as pltpu
from jax.experimental.pallas import tpu_sc as plsc

VOCAB = 1000
BATCH = 16384
HIST = 200
D = 32
B = BATCH * HIST
NC = 2
NS = 16
NW = NC * NS
TOK_PER_W = B // NW
CHUNK = 1024
NITER = TOK_PER_W // CHUNK
NBUF = 2
NBODY = NITER // NBUF

SROWS = 4
SS = SROWS * 128
CC = CHUNK - SS
NGRP = CC // 16

_mesh = plsc.VectorSubcoreMesh(core_axis_name="c", subcore_axis_name="s")

_SPLAT_DNUMS = lax.GatherDimensionNumbers(
    offset_dims=(), collapsed_slice_dims=(0,), start_index_map=(0,)
)


def _lane_splat(vec, t):
    idx = jnp.full((16, 1), t, jnp.int32)
    return lax.gather(
        vec,
        idx,
        _SPLAT_DNUMS,
        (1,),
        mode=lax.GatherScatterMode.PROMISE_IN_BOUNDS,
    )


@functools.partial(
    pl.kernel,
    mesh=_mesh,
    compiler_params=pltpu.CompilerParams(
        needs_layout_passes=False, use_tc_tiling_on_sc=False
    ),
    out_type=jax.ShapeDtypeStruct((B, D), jnp.float32),
    scratch_types=[
        pltpu.VMEM_SHARED((VOCAB, D), jnp.float32),
        pltpu.VMEM((VOCAB, D), jnp.float32),
        [pltpu.VMEM((CHUNK,), jnp.int32)] * NBUF,
        [pltpu.VMEM((SS, D), jnp.float32)] * NBUF,
        [pltpu.VMEM((CC, D), jnp.float32)] * NBUF,
        pltpu.SemaphoreType.DMA,
        [pltpu.SemaphoreType.DMA] * NBUF,
        [pltpu.SemaphoreType.DMA] * NBUF,
        [pltpu.SemaphoreType.DMA] * NBUF,
    ],
)
def _emb_lookup(
    idx_hbm, w_hbm, out_hbm,
    w_sh, w_loc, idx_v, rows_s, rows_c,
    gsem, ssems, csems, isems,
):
    sid = lax.axis_index("s")
    wid = sid * NC + lax.axis_index("c")
    base = wid * TOK_PER_W

    pltpu.sync_copy(w_hbm, w_loc)

    @pl.when(sid == 0)
    def _stage_table():
        pltpu.sync_copy(w_hbm, w_sh)

    plsc.subcore_barrier()

    lane16 = lax.iota(jnp.int32, 16)

    def body(s, carry):
        for k in range(NBUF):
            i = s * NBUF + k
            t0 = base + i * CHUNK
            idx_b = idx_v[k]
            rs_b = rows_s[k]
            rc_b = rows_c[k]

            @pl.when(s > 0)
            def _drain_prev():
                pltpu.make_async_copy(
                    rs_b, out_hbm.at[pl.ds(t0, SS)], ssems[k]
                ).wait()
                pltpu.make_async_copy(
                    rc_b, out_hbm.at[pl.ds(t0, CC)], csems[k]
                ).wait()
                pltpu.make_async_copy(
                    idx_hbm.at[pl.ds(t0, CHUNK)], idx_b, isems[k]
                ).wait()

            @pl.when(s == 0)
            def _prime_idx():
                pltpu.sync_copy(idx_hbm.at[pl.ds(t0, CHUNK)], idx_b)

            copies = [
                pltpu.async_copy(
                    w_sh.at[idx_b.at[pl.ds(j * 128, 128)]],
                    rs_b.at[pl.ds(j * 128, 128)],
                    gsem,
                )
                for j in range(SROWS)
            ]

            @plsc.parallel_loop(0, NGRP, unroll=4)
            def _group(g):
                tok16 = idx_b[pl.ds(SS + g * 16, 16)]
                for t in range(16):
                    bs = _lane_splat(tok16, t)
                    v0 = plsc.load_gather(w_loc, [bs, lane16])
                    v1 = plsc.load_gather(w_loc, [bs, lane16 + 16])
                    tloc = g * 16 + t
                    rc_b[tloc, pl.ds(0, 16)] = v0
                    rc_b[tloc, pl.ds(16, 16)] = v1

            for cp in copies:
                cp.wait()

            @pl.when(s < NBODY - 1)
            def _prefetch_idx():
                t0n = t0 + NBUF * CHUNK
                pltpu.async_copy(idx_hbm.at[pl.ds(t0n, CHUNK)], idx_b, isems[k])

            pltpu.async_copy(rs_b, out_hbm.at[pl.ds(t0, SS)], ssems[k])
            pltpu.async_copy(rc_b, out_hbm.at[pl.ds(t0 + SS, CC)], csems[k])
        return carry

    lax.fori_loop(0, NBODY, body, 0)

    for k in range(NBUF):
        pltpu.make_async_copy(
            rows_s[k], out_hbm.at[pl.ds(base, SS)], ssems[k]
        ).wait()
        pltpu.make_async_copy(
            rows_c[k], out_hbm.at[pl.ds(base, CC)], csems[k]
        ).wait()


def kernel(token_idxs, weight):
    idx = token_idxs.reshape(B)
    out = _emb_lookup(idx, weight)
    return out.reshape(BATCH, HIST, D)


# speedup vs baseline: 2.6625x; 2.6625x over previous
import functools

import jax
import jax.numpy as jnp
from jax import lax
from jax.experimental import pallas as pl
from jax.experimental.pallas import tpu as pltpu
from jax.experimental.pallas import tpu_sc as plsc

VOCAB = 1000
BATCH = 16384
HIST = 200
D = 32
NC = 2
NS = 16
NW = NC * NS
BB = BATCH // NW
BBP = BB + 1
NGRP = BB // 16
NBUF = 2
NBODY = HIST // NBUF

_mesh = plsc.VectorSubcoreMesh(core_axis_name="c", subcore_axis_name="s")

_SPLAT_DNUMS = lax.GatherDimensionNumbers(
    offset_dims=(), collapsed_slice_dims=(0,), start_index_map=(0,)
)


def _lane_splat(vec, t):
    idx = jnp.full((16, 1), t, jnp.int32)
    return lax.gather(
        vec,
        idx,
        _SPLAT_DNUMS,
        (1,),
        mode=lax.GatherScatterMode.PROMISE_IN_BOUNDS,
    )


@functools.partial(
    pl.kernel,
    mesh=_mesh,
    compiler_params=pltpu.CompilerParams(
        needs_layout_passes=False, use_tc_tiling_on_sc=False
    ),
    out_type=jax.ShapeDtypeStruct((HIST, D, BATCH), jnp.float32),
    scratch_types=[
        pltpu.VMEM((VOCAB, D), jnp.float32),
        [pltpu.VMEM((BB,), jnp.int32)] * NBUF,
        [pltpu.VMEM((D, BBP), jnp.float32)] * NBUF,
        [pltpu.SemaphoreType.DMA] * NBUF,
        [pltpu.SemaphoreType.DMA] * NBUF,
    ],
)
def _emb_lookup(idxT_hbm, w_hbm, out_hbm, w_loc, idx_v, cols_v, ssems, isems):
    wid = lax.axis_index("s") * NC + lax.axis_index("c")
    b0 = wid * BB

    pltpu.sync_copy(w_hbm, w_loc)

    lane16 = lax.iota(jnp.int32, 16)
    lane16h = lane16 + 16

    def body(s, carry):
        for k in range(NBUF):
            h = s * NBUF + k
            idx_b = idx_v[k]
            col_b = cols_v[k]
            col_store = col_b.at[:, pl.ds(0, BB)]

            @pl.when(s > 0)
            def _drain_prev():
                pltpu.make_async_copy(
                    col_store, out_hbm.at[h, :, pl.ds(b0, BB)], ssems[k]
                ).wait()
                pltpu.make_async_copy(
                    idxT_hbm.at[h, pl.ds(b0, BB)], idx_b, isems[k]
                ).wait()

            @pl.when(s == 0)
            def _prime_idx():
                pltpu.sync_copy(idxT_hbm.at[h, pl.ds(b0, BB)], idx_b)

            @plsc.parallel_loop(0, NGRP, unroll=1)
            def _group(g):
                tok16 = idx_b[pl.ds(g * 16, 16)]
                for t in range(16):
                    bs = _lane_splat(tok16, t)
                    v0 = plsc.load_gather(w_loc, [bs, lane16])
                    v1 = plsc.load_gather(w_loc, [bs, lane16h])
                    tcol = jnp.full((16,), g * 16 + t, jnp.int32)
                    plsc.store_scatter(col_b, [lane16, tcol], v0)
                    plsc.store_scatter(col_b, [lane16h, tcol], v1)

            @pl.when(s < NBODY - 1)
            def _prefetch_idx():
                pltpu.async_copy(
                    idxT_hbm.at[h + NBUF, pl.ds(b0, BB)], idx_b, isems[k]
                )

            pltpu.async_copy(
                col_store, out_hbm.at[h, :, pl.ds(b0, BB)], ssems[k]
            )
        return carry

    lax.fori_loop(0, NBODY, body, 0)

    for k in range(NBUF):
        pltpu.make_async_copy(
            cols_v[k].at[:, pl.ds(0, BB)],
            out_hbm.at[0, :, pl.ds(b0, BB)],
            ssems[k],
        ).wait()


def kernel(token_idxs, weight):
    idx_t = jnp.transpose(token_idxs)
    out = _emb_lookup(idx_t, weight)
    return jnp.transpose(out, (2, 0, 1))
